# Initial kernel scaffold; baseline (speedup 1.0000x reference)
import jax
import jax.numpy as jnp
from jax.experimental import pallas as pl


def kernel(x, W1, g1, b1, W2, g2, b2, W3, g3, b3):
    raise NotImplementedError("write your pallas kernel here")


# speedup vs baseline: 29.5572x; 29.5572x over previous
import jax
import jax.numpy as jnp
from jax.experimental import pallas as pl

_B, _N, _F, _C = 8, 2048, 64, 64
_M = _B * _N


def _mlp_bn_kernel(x_ref, w1_ref, g1_ref, b1_ref, w2_ref, g2_ref, b2_ref,
                   w3_ref, g3_ref, b3_ref, out_ref):
    a = x_ref[...].reshape(_M, _F)

    def layer(h, w_ref, g_ref, b_ref):
        hm = jax.lax.dot_general(h, w_ref[...], (((1,), (1,)), ((), ())),
                                 preferred_element_type=jnp.float32)
        mean = jnp.sum(hm, axis=0, keepdims=True) * (1.0 / _M)
        ex2 = jnp.sum(hm * hm, axis=0, keepdims=True) * (1.0 / _M)
        var = ex2 - mean * mean
        hn = (hm - mean) * jax.lax.rsqrt(var + 1e-5) * g_ref[...] + b_ref[...]
        return hn * 0.5 * (1.0 + jax.lax.erf(hn * 0.7071067811865476))

    a = layer(a, w1_ref, g1_ref, b1_ref)
    a = layer(a, w2_ref, g2_ref, b2_ref)
    a = layer(a, w3_ref, g3_ref, b3_ref)
    out_ref[...] = jnp.transpose(a.reshape(_B, _N, _C), (0, 2, 1))


def kernel(x, W1, g1, b1, W2, g2, b2, W3, g3, b3):
    return pl.pallas_call(
        _mlp_bn_kernel,
        out_shape=jax.ShapeDtypeStruct((_B, _C, _N), jnp.float32),
    )(x, W1, g1.reshape(1, _C), b1.reshape(1, _C),
      W2, g2.reshape(1, _C), b2.reshape(1, _C),
      W3, g3.reshape(1, _C), b3.reshape(1, _C))


# speedup vs baseline: 30.7875x; 1.0416x over previous
import jax
import jax.numpy as jnp
from jax.experimental import pallas as pl

_B, _N, _F, _C = 8, 2048, 64, 64
_M = _B * _N
_P = 4
_R = _M // _P
_L = _P * _C
_INV_SQRT2 = 0.7071067811865476
_POST = 2.0 ** 0.5 / 2.0


def _mlp_bn_kernel(x_ref, w1_ref, g1_ref, b1_ref, w2_ref, g2_ref, b2_ref,
                   w3_ref, g3_ref, b3_ref, out_ref):
    x3 = x_ref[...]
    a = jnp.concatenate([x3[i] for i in range(_P)], axis=1)

    def layer(h, w_ref, g_ref, b_ref, post):
        hm = jax.lax.dot_general(h, w_ref[...], (((1,), (0,)), ((), ())),
                                 preferred_element_type=jnp.float32)
        s1 = jnp.sum(hm, axis=0, keepdims=True)
        s2 = jnp.sum(hm * hm, axis=0, keepdims=True)
        s1 = sum(s1[:, i * _C:(i + 1) * _C] for i in range(_P)) * (1.0 / _M)
        s2 = sum(s2[:, i * _C:(i + 1) * _C] for i in range(_P)) * (1.0 / _M)
        var = s2 - s1 * s1
        scale = jax.lax.rsqrt(var + 1e-5) * g_ref[...] * _INV_SQRT2
        offset = b_ref[...] * _INV_SQRT2 - s1 * scale
        scale = jnp.concatenate([scale] * _P, axis=1)
        offset = jnp.concatenate([offset] * _P, axis=1)
        t = hm * scale + offset
        out = t * (1.0 + jax.lax.erf(t))
        return out * post if post != 1.0 else out

    a = layer(a, w1_ref, g1_ref, b1_ref, 1.0)
    a = layer(a, w2_ref, g2_ref, b2_ref, 1.0)
    a = layer(a, w3_ref, g3_ref, b3_ref, _POST)
    for q in range(_P):
        for p in range(_R // _N):
            out_ref[(_R // _N) * q + p, :, :] = jnp.transpose(
                a[p * _N:(p + 1) * _N, q * _C:(q + 1) * _C], (1, 0))


def _blockdiag(W, pre):
    return jnp.kron(jnp.eye(_P, dtype=W.dtype), W.T * pre)


def kernel(x, W1, g1, b1, W2, g2, b2, W3, g3, b3):
    xp = x.reshape(_P, _R, _F)
    return pl.pallas_call(
        _mlp_bn_kernel,
        out_shape=jax.ShapeDtypeStruct((_B, _C, _N), jnp.float32),
    )(xp, _blockdiag(W1, 1.0), g1.reshape(1, _C), b1.reshape(1, _C),
      _blockdiag(W2, _POST), g2.reshape(1, _C), b2.reshape(1, _C),
      _blockdiag(W3, _POST), g3.reshape(1, _C), b3.reshape(1, _C))
